# baseline (speedup 1.0000x reference)
import functools

import jax
import jax.numpy as jnp
from jax import lax
from jax.experimental import pallas as pl
from jax.experimental.pallas import tpu as pltpu
from jax.experimental.pallas import tpu_sc as plsc

NC = 2
NS = 16
NW = NC * NS
K = 128
ZR = 64

_SC_PARAMS = pltpu.CompilerParams(use_tc_tiling_on_sc=False)


@functools.cache
def _mesh():
    return plsc.VectorSubcoreMesh(core_axis_name="c", subcore_axis_name="s",
                                  num_cores=NC, num_subcores=NS)


def _sc_degree(dst_r, n_pad):
    nch = dst_r.shape[1]
    r = n_pad // NS

    @functools.partial(
        pl.kernel,
        out_type=jax.ShapeDtypeStruct((NC, n_pad, 16), jnp.float32),
        mesh=_mesh(),
        scratch_types=[
            pltpu.VMEM((nch, K), jnp.int32),
            pltpu.VMEM((K, 16), jnp.float32),
            pltpu.VMEM((ZR, 16), jnp.float32),
            pltpu.VMEM_SHARED((n_pad, 16), jnp.float32),
        ],
        compiler_params=_SC_PARAMS,
    )
    def deg_kernel(dst_hbm, out_hbm, didx, obuf, zbuf, accum):
        c = lax.axis_index("c")
        s = lax.axis_index("s")
        wid = s * NC + c

        pltpu.sync_copy(dst_hbm.at[wid], didx)

        @pl.loop(0, ZR)
        def _(i):
            zbuf[i] = jnp.zeros((16,), jnp.float32)

        @pl.loop(0, K)
        def _(i):
            obuf[i] = jnp.ones((16,), jnp.float32)

        @pl.loop(0, r, step=ZR)
        def _(rr):
            pltpu.sync_copy(zbuf, accum.at[pl.ds(s * r + rr, ZR)])

        plsc.subcore_barrier()

        @pl.loop(0, nch)
        def _(t):
            pltpu.sync_copy(obuf, accum.at[didx.at[t]], add=True)

        plsc.subcore_barrier()
        pltpu.sync_copy(accum.at[pl.ds(s * r, r)], out_hbm.at[c].at[pl.ds(s * r, r)])

    return deg_kernel(dst_r)


def _sc_aggregate(hp, src_f, dst_f, n_pad, d, nbuf):
    e_pad = src_f.shape[0]
    ep = e_pad // NW
    r = n_pad // NS

    scratch = (
        [pltpu.VMEM((K,), jnp.int32) for _ in range(2 * nbuf)]
        + [pltpu.VMEM((K, d), jnp.float32) for _ in range(nbuf)]
        + [pltpu.VMEM((ZR, d), jnp.float32),
           pltpu.VMEM_SHARED((n_pad, d), jnp.float32)]
        + [pltpu.SemaphoreType.DMA for _ in range(nbuf)]
    )

    @functools.partial(
        pl.kernel,
        out_type=jax.ShapeDtypeStruct((NC, n_pad, d), jnp.float32),
        mesh=_mesh(),
        scratch_types=scratch,
        compiler_params=_SC_PARAMS,
    )
    def agg_kernel(hp_hbm, src_hbm, dst_hbm, out_hbm, *refs):
        sidx = refs[:nbuf]
        didx = refs[nbuf:2 * nbuf]
        gbuf = refs[2 * nbuf:3 * nbuf]
        zbuf = refs[3 * nbuf]
        accum = refs[3 * nbuf + 1]
        sem = refs[3 * nbuf + 2:]

        c = lax.axis_index("c")
        s = lax.axis_index("s")
        wid = s * NC + c

        @pl.loop(0, ZR)
        def _(i):
            @pl.loop(0, d, step=16)
            def _(j):
                zbuf[i, pl.ds(j, 16)] = jnp.zeros((16,), jnp.float32)

        @pl.loop(0, r, step=ZR)
        def _(rr):
            pltpu.sync_copy(zbuf, accum.at[pl.ds(s * r + rr, ZR)])

        plsc.subcore_barrier()

        base = wid * ep

        for b in range(nbuf - 1):
            pltpu.sync_copy(src_hbm.at[pl.ds(base + b * K, K)], sidx[b])
            pltpu.sync_copy(dst_hbm.at[pl.ds(base + b * K, K)], didx[b])
            pltpu.async_copy(hp_hbm.at[sidx[b]], gbuf[b], sem[b])

        @pl.loop(0, ep, step=nbuf * K)
        def _(t):
            for b in range(nbuf):
                cur = t + b * K
                pre = cur + (nbuf - 1) * K
                sn = (b + nbuf - 1) % nbuf

                @pl.when(pre < ep)
                def _(pre=pre, sn=sn):
                    pltpu.sync_copy(src_hbm.at[pl.ds(base + pre, K)], sidx[sn])
                    pltpu.sync_copy(dst_hbm.at[pl.ds(base + pre, K)], didx[sn])
                    pltpu.async_copy(hp_hbm.at[sidx[sn]], gbuf[sn], sem[sn])

                pltpu.make_async_copy(hp_hbm.at[sidx[b]], gbuf[b], sem[b]).wait()
                pltpu.sync_copy(gbuf[b], accum.at[didx[b]], add=True)

        plsc.subcore_barrier()
        pltpu.sync_copy(accum.at[pl.ds(s * r, r)], out_hbm.at[c].at[pl.ds(s * r, r)])

    return agg_kernel(hp, src_f, dst_f)


def _dinv_from(dp_ref):
    deg = dp_ref[0, :, :1] + dp_ref[1, :, :1]
    return jnp.where(deg > 0, lax.rsqrt(deg), 0.0)


def _tc_k1(x_pad, W1, degp, blk):
    n_pad, d_in = x_pad.shape
    d_hid = W1.shape[1]

    def body(x_ref, w_ref, dp_ref, o_ref):
        dinv = _dinv_from(dp_ref)
        o_ref[...] = jnp.dot(x_ref[...], w_ref[...],
                             preferred_element_type=jnp.float32) * dinv

    return pl.pallas_call(
        body,
        grid=(n_pad // blk,),
        in_specs=[
            pl.BlockSpec((blk, d_in), lambda i: (i, 0)),
            pl.BlockSpec((d_in, d_hid), lambda i: (0, 0)),
            pl.BlockSpec((NC, blk, 16), lambda i: (0, i, 0)),
        ],
        out_specs=pl.BlockSpec((blk, d_hid), lambda i: (i, 0)),
        out_shape=jax.ShapeDtypeStruct((n_pad, d_hid), jnp.float32),
    )(x_pad, W1, degp)


def _tc_k2(p, degp, b1, W2, blk):
    n_pad, d_hid = p.shape[1], p.shape[2]
    d_emb = W2.shape[1]

    def body(p_ref, dp_ref, b_ref, w_ref, o_ref):
        dinv = _dinv_from(dp_ref)
        h1 = jnp.maximum((p_ref[0] + p_ref[1]) * dinv + b_ref[...], 0.0)
        o_ref[...] = jnp.dot(h1, w_ref[...],
                             preferred_element_type=jnp.float32) * dinv

    return pl.pallas_call(
        body,
        grid=(n_pad // blk,),
        in_specs=[
            pl.BlockSpec((NC, blk, d_hid), lambda i: (0, i, 0)),
            pl.BlockSpec((NC, blk, 16), lambda i: (0, i, 0)),
            pl.BlockSpec((1, d_hid), lambda i: (0, 0)),
            pl.BlockSpec((d_hid, d_emb), lambda i: (0, 0)),
        ],
        out_specs=pl.BlockSpec((blk, d_emb), lambda i: (i, 0)),
        out_shape=jax.ShapeDtypeStruct((n_pad, d_emb), jnp.float32),
    )(p, degp, b1.reshape(1, -1), W2)


def _tc_k3(q, degp, b2, Wc_pad, bc_pad, blk):
    n_pad, d_emb = q.shape[1], q.shape[2]
    n_out = Wc_pad.shape[1]

    def body(q_ref, dp_ref, b_ref, w_ref, bc_ref, o_ref):
        dinv = _dinv_from(dp_ref)
        h2 = jnp.maximum((q_ref[0] + q_ref[1]) * dinv + b_ref[...], 0.0)
        o_ref[...] = jnp.dot(h2, w_ref[...],
                             preferred_element_type=jnp.float32) + bc_ref[...]

    return pl.pallas_call(
        body,
        grid=(n_pad // blk,),
        in_specs=[
            pl.BlockSpec((NC, blk, d_emb), lambda i: (0, i, 0)),
            pl.BlockSpec((NC, blk, 16), lambda i: (0, i, 0)),
            pl.BlockSpec((1, d_emb), lambda i: (0, 0)),
            pl.BlockSpec((d_emb, n_out), lambda i: (0, 0)),
            pl.BlockSpec((1, n_out), lambda i: (0, 0)),
        ],
        out_specs=pl.BlockSpec((blk, n_out), lambda i: (i, 0)),
        out_shape=jax.ShapeDtypeStruct((n_pad, n_out), jnp.float32),
    )(q, degp, b2.reshape(1, -1), Wc_pad, bc_pad)


def kernel(x, edge_index, W1, b1, W2, b2, Wc, bc):
    n, d_in = x.shape
    d_hid = W1.shape[1]
    d_emb = W2.shape[1]
    n_cls = Wc.shape[1]
    e = edge_index.shape[1]
    e_tot = e + n

    ep = -(-e_tot // (NW * 4 * K)) * 4 * K
    e_pad = ep * NW
    nch = ep // K
    n_pad = -(-(n + 1) // (NS * ZR)) * (NS * ZR)

    idt = edge_index.dtype
    loop_idx = jnp.arange(n, dtype=idt)
    pad_idx = n + jnp.arange(e_pad - e_tot, dtype=idt) % (n_pad - n)
    src_f = jnp.concatenate([edge_index[0], loop_idx, pad_idx])
    dst_f = jnp.concatenate([edge_index[1], loop_idx, pad_idx])
    dst_r = dst_f.reshape(NW, nch, K)

    x_pad = jnp.pad(x.astype(jnp.float32), ((0, n_pad - n), (0, 0)))
    wc_pad = jnp.pad(Wc, ((0, 0), (0, 128 - n_cls)))
    bc_pad = jnp.pad(bc, (0, 128 - n_cls)).reshape(1, -1)

    blk = 1024
    degp = _sc_degree(dst_r, n_pad)
    hp = _tc_k1(x_pad, W1, degp, blk)
    p = _sc_aggregate(hp, src_f, dst_f, n_pad, d_hid, 2)
    h2p = _tc_k2(p, degp, b1, W2, blk)
    q = _sc_aggregate(h2p, src_f, dst_f, n_pad, d_emb, 4)
    outp = _tc_k3(q, degp, b2, wc_pad, bc_pad, blk)
    return outp[:n, :n_cls]


# speedup vs baseline: 1.0161x; 1.0161x over previous
import functools

import jax
import jax.numpy as jnp
from jax import lax
from jax.experimental import pallas as pl
from jax.experimental.pallas import tpu as pltpu
from jax.experimental.pallas import tpu_sc as plsc

NC = 2
NS = 16
NW = NC * NS
K = 128
ZR = 64

_SC_PARAMS = pltpu.CompilerParams(use_tc_tiling_on_sc=False)


@functools.cache
def _mesh():
    return plsc.VectorSubcoreMesh(core_axis_name="c", subcore_axis_name="s",
                                  num_cores=NC, num_subcores=NS)


def _sc_degree(dst_r, n_pad):
    nch = dst_r.shape[1]
    r = n_pad // NS

    @functools.partial(
        pl.kernel,
        out_type=jax.ShapeDtypeStruct((NC, n_pad, 16), jnp.float32),
        mesh=_mesh(),
        scratch_types=[
            pltpu.VMEM((nch, K), jnp.int32),
            pltpu.VMEM((K, 16), jnp.float32),
            pltpu.VMEM((ZR, 16), jnp.float32),
            pltpu.VMEM_SHARED((n_pad, 16), jnp.float32),
        ],
        compiler_params=_SC_PARAMS,
    )
    def deg_kernel(dst_hbm, out_hbm, didx, obuf, zbuf, accum):
        c = lax.axis_index("c")
        s = lax.axis_index("s")
        wid = s * NC + c

        pltpu.sync_copy(dst_hbm.at[wid], didx)

        @pl.loop(0, ZR)
        def _(i):
            zbuf[i] = jnp.zeros((16,), jnp.float32)

        @pl.loop(0, K)
        def _(i):
            obuf[i] = jnp.ones((16,), jnp.float32)

        @pl.loop(0, r, step=ZR)
        def _(rr):
            pltpu.sync_copy(zbuf, accum.at[pl.ds(s * r + rr, ZR)])

        plsc.subcore_barrier()

        @pl.loop(0, nch)
        def _(t):
            pltpu.sync_copy(obuf, accum.at[didx.at[t]], add=True)

        plsc.subcore_barrier()
        pltpu.sync_copy(accum.at[pl.ds(s * r, r)], out_hbm.at[c].at[pl.ds(s * r, r)])

    return deg_kernel(dst_r)


def _sc_aggregate(hp, src_f, dst_f, n_pad, d, nbuf):
    e_pad = src_f.shape[0]
    ep = e_pad // NW
    r = n_pad // NS

    scratch = (
        [pltpu.VMEM((K,), jnp.int32) for _ in range(2 * nbuf)]
        + [pltpu.VMEM((K, d), jnp.float32) for _ in range(nbuf)]
        + [pltpu.VMEM((ZR, d), jnp.float32),
           pltpu.VMEM_SHARED((n_pad, d), jnp.float32)]
        + [pltpu.SemaphoreType.DMA for _ in range(nbuf)]
    )

    @functools.partial(
        pl.kernel,
        out_type=jax.ShapeDtypeStruct((NC, n_pad, d), jnp.float32),
        mesh=_mesh(),
        scratch_types=scratch,
        compiler_params=_SC_PARAMS,
    )
    def agg_kernel(hp_hbm, src_hbm, dst_hbm, out_hbm, *refs):
        sidx = refs[:nbuf]
        didx = refs[nbuf:2 * nbuf]
        gbuf = refs[2 * nbuf:3 * nbuf]
        zbuf = refs[3 * nbuf]
        accum = refs[3 * nbuf + 1]
        sem = refs[3 * nbuf + 2:]

        c = lax.axis_index("c")
        s = lax.axis_index("s")
        wid = s * NC + c

        @pl.loop(0, ZR)
        def _(i):
            @pl.loop(0, d, step=16)
            def _(j):
                zbuf[i, pl.ds(j, 16)] = jnp.zeros((16,), jnp.float32)

        @pl.loop(0, r, step=ZR)
        def _(rr):
            pltpu.sync_copy(zbuf, accum.at[pl.ds(s * r + rr, ZR)])

        plsc.subcore_barrier()

        base = wid * ep

        for b in range(nbuf - 1):
            pltpu.sync_copy(src_hbm.at[pl.ds(base + b * K, K)], sidx[b])
            pltpu.sync_copy(dst_hbm.at[pl.ds(base + b * K, K)], didx[b])
            pltpu.async_copy(hp_hbm.at[sidx[b]], gbuf[b], sem[b])

        @pl.loop(0, ep, step=nbuf * K)
        def _(t):
            for b in range(nbuf):
                cur = t + b * K
                pre = cur + (nbuf - 1) * K
                sn = (b + nbuf - 1) % nbuf

                @pl.when(pre < ep)
                def _(pre=pre, sn=sn):
                    pltpu.sync_copy(src_hbm.at[pl.ds(base + pre, K)], sidx[sn])
                    pltpu.sync_copy(dst_hbm.at[pl.ds(base + pre, K)], didx[sn])
                    pltpu.async_copy(hp_hbm.at[sidx[sn]], gbuf[sn], sem[sn])

                pltpu.make_async_copy(hp_hbm.at[sidx[b]], gbuf[b], sem[b]).wait()
                pltpu.sync_copy(gbuf[b], accum.at[didx[b]], add=True)

        plsc.subcore_barrier()
        pltpu.sync_copy(accum.at[pl.ds(s * r, r)], out_hbm.at[c].at[pl.ds(s * r, r)])

    return agg_kernel(hp, src_f, dst_f)


def _dinv_from(dp_ref):
    deg = dp_ref[0, :, :1] + dp_ref[1, :, :1]
    return jnp.where(deg > 0, lax.rsqrt(deg), 0.0)


def _tc_k1(x_pad, W1, degp, blk):
    n_pad, d_in = x_pad.shape
    d_hid = W1.shape[1]

    def body(x_ref, w_ref, dp_ref, o_ref):
        dinv = _dinv_from(dp_ref)
        o_ref[...] = jnp.dot(x_ref[...], w_ref[...],
                             preferred_element_type=jnp.float32) * dinv

    return pl.pallas_call(
        body,
        grid=(n_pad // blk,),
        in_specs=[
            pl.BlockSpec((blk, d_in), lambda i: (i, 0)),
            pl.BlockSpec((d_in, d_hid), lambda i: (0, 0)),
            pl.BlockSpec((NC, blk, 16), lambda i: (0, i, 0)),
        ],
        out_specs=pl.BlockSpec((blk, d_hid), lambda i: (i, 0)),
        out_shape=jax.ShapeDtypeStruct((n_pad, d_hid), jnp.float32),
    )(x_pad, W1, degp)


def _tc_k2(p, degp, b1, W2, blk):
    n_pad, d_hid = p.shape[1], p.shape[2]
    d_emb = W2.shape[1]

    def body(p_ref, dp_ref, b_ref, w_ref, o_ref):
        dinv = _dinv_from(dp_ref)
        h1 = jnp.maximum((p_ref[0] + p_ref[1]) * dinv + b_ref[...], 0.0)
        o_ref[...] = jnp.dot(h1, w_ref[...],
                             preferred_element_type=jnp.float32) * dinv

    return pl.pallas_call(
        body,
        grid=(n_pad // blk,),
        in_specs=[
            pl.BlockSpec((NC, blk, d_hid), lambda i: (0, i, 0)),
            pl.BlockSpec((NC, blk, 16), lambda i: (0, i, 0)),
            pl.BlockSpec((1, d_hid), lambda i: (0, 0)),
            pl.BlockSpec((d_hid, d_emb), lambda i: (0, 0)),
        ],
        out_specs=pl.BlockSpec((blk, d_emb), lambda i: (i, 0)),
        out_shape=jax.ShapeDtypeStruct((n_pad, d_emb), jnp.float32),
    )(p, degp, b1.reshape(1, -1), W2)


def _tc_k3(q, degp, b2, Wc_pad, bc_pad, blk):
    n_pad, d_emb = q.shape[1], q.shape[2]
    n_out = Wc_pad.shape[1]

    def body(q_ref, dp_ref, b_ref, w_ref, bc_ref, o_ref):
        dinv = _dinv_from(dp_ref)
        h2 = jnp.maximum((q_ref[0] + q_ref[1]) * dinv + b_ref[...], 0.0)
        o_ref[...] = jnp.dot(h2, w_ref[...],
                             preferred_element_type=jnp.float32) + bc_ref[...]

    return pl.pallas_call(
        body,
        grid=(n_pad // blk,),
        in_specs=[
            pl.BlockSpec((NC, blk, d_emb), lambda i: (0, i, 0)),
            pl.BlockSpec((NC, blk, 16), lambda i: (0, i, 0)),
            pl.BlockSpec((1, d_emb), lambda i: (0, 0)),
            pl.BlockSpec((d_emb, n_out), lambda i: (0, 0)),
            pl.BlockSpec((1, n_out), lambda i: (0, 0)),
        ],
        out_specs=pl.BlockSpec((blk, n_out), lambda i: (i, 0)),
        out_shape=jax.ShapeDtypeStruct((n_pad, n_out), jnp.float32),
    )(q, degp, b2.reshape(1, -1), Wc_pad, bc_pad)


def kernel(x, edge_index, W1, b1, W2, b2, Wc, bc):
    n, d_in = x.shape
    d_hid = W1.shape[1]
    d_emb = W2.shape[1]
    n_cls = Wc.shape[1]
    e = edge_index.shape[1]
    e_tot = e + n

    ep = -(-e_tot // (NW * 2 * K)) * 2 * K
    e_pad = ep * NW
    nch = ep // K
    n_pad = -(-(n + 1) // (NS * ZR)) * (NS * ZR)

    idt = edge_index.dtype
    loop_idx = jnp.arange(n, dtype=idt)
    pad_idx = n + jnp.arange(e_pad - e_tot, dtype=idt) % (n_pad - n)
    src_f = jnp.concatenate([edge_index[0], loop_idx, pad_idx])
    dst_f = jnp.concatenate([edge_index[1], loop_idx, pad_idx])
    dst_r = dst_f.reshape(NW, nch, K)

    x_pad = jnp.pad(x.astype(jnp.float32), ((0, n_pad - n), (0, 0)))
    wc_pad = jnp.pad(Wc, ((0, 0), (0, 128 - n_cls)))
    bc_pad = jnp.pad(bc, (0, 128 - n_cls)).reshape(1, -1)

    blk = 1024
    degp = _sc_degree(dst_r, n_pad)
    hp = _tc_k1(x_pad, W1, degp, blk)
    p = _sc_aggregate(hp, src_f, dst_f, n_pad, d_hid, 2)
    h2p = _tc_k2(p, degp, b1, W2, blk)
    q = _sc_aggregate(h2p, src_f, dst_f, n_pad, d_emb, 2)
    outp = _tc_k3(q, degp, b2, wc_pad, bc_pad, blk)
    return outp[:n, :n_cls]


# speedup vs baseline: 1.0286x; 1.0123x over previous
import functools

import jax
import jax.numpy as jnp
from jax import lax
from jax.experimental import pallas as pl
from jax.experimental.pallas import tpu as pltpu
from jax.experimental.pallas import tpu_sc as plsc

NC = 2
NS = 16
NW = NC * NS
K = 128
ZR = 64

_SC_PARAMS = pltpu.CompilerParams(use_tc_tiling_on_sc=False)


@functools.cache
def _mesh():
    return plsc.VectorSubcoreMesh(core_axis_name="c", subcore_axis_name="s",
                                  num_cores=NC, num_subcores=NS)


def _sc_degree(dst_r, n_pad):
    nch = dst_r.shape[1]
    r = n_pad // NS

    @functools.partial(
        pl.kernel,
        out_type=jax.ShapeDtypeStruct((NC, n_pad, 16), jnp.float32),
        mesh=_mesh(),
        scratch_types=[
            pltpu.VMEM((nch, K), jnp.int32),
            pltpu.VMEM((K, 16), jnp.float32),
            pltpu.VMEM((ZR, 16), jnp.float32),
            pltpu.VMEM_SHARED((n_pad, 16), jnp.float32),
        ],
        compiler_params=_SC_PARAMS,
    )
    def deg_kernel(dst_hbm, out_hbm, didx, obuf, zbuf, accum):
        c = lax.axis_index("c")
        s = lax.axis_index("s")
        wid = s * NC + c

        pltpu.sync_copy(dst_hbm.at[wid], didx)

        @pl.loop(0, ZR)
        def _(i):
            zbuf[i] = jnp.zeros((16,), jnp.float32)

        @pl.loop(0, K)
        def _(i):
            obuf[i] = jnp.ones((16,), jnp.float32)

        @pl.loop(0, r, step=ZR)
        def _(rr):
            pltpu.sync_copy(zbuf, accum.at[pl.ds(s * r + rr, ZR)])

        plsc.subcore_barrier()

        @pl.loop(0, nch)
        def _(t):
            pltpu.sync_copy(obuf, accum.at[didx.at[t]], add=True)

        plsc.subcore_barrier()
        pltpu.sync_copy(accum.at[pl.ds(s * r, r)], out_hbm.at[c].at[pl.ds(s * r, r)])

    return deg_kernel(dst_r)


def _sc_aggregate(hp, src_f, dst_f, n_pad, d, nbuf):
    e_pad = src_f.shape[0]
    ep = e_pad // NW
    r = n_pad // NS

    scratch = (
        [pltpu.VMEM((K,), jnp.int32) for _ in range(2 * nbuf)]
        + [pltpu.VMEM((K, d), jnp.float32) for _ in range(nbuf)]
        + [pltpu.VMEM((ZR, d), jnp.float32),
           pltpu.VMEM_SHARED((n_pad, d), jnp.float32)]
        + [pltpu.SemaphoreType.DMA for _ in range(nbuf)]
    )

    @functools.partial(
        pl.kernel,
        out_type=jax.ShapeDtypeStruct((NC, n_pad, d), jnp.float32),
        mesh=_mesh(),
        scratch_types=scratch,
        compiler_params=_SC_PARAMS,
    )
    def agg_kernel(hp_hbm, src_hbm, dst_hbm, out_hbm, *refs):
        sidx = refs[:nbuf]
        didx = refs[nbuf:2 * nbuf]
        gbuf = refs[2 * nbuf:3 * nbuf]
        zbuf = refs[3 * nbuf]
        accum = refs[3 * nbuf + 1]
        sem = refs[3 * nbuf + 2:]

        c = lax.axis_index("c")
        s = lax.axis_index("s")
        wid = s * NC + c

        @pl.loop(0, ZR)
        def _(i):
            @pl.loop(0, d, step=16)
            def _(j):
                zbuf[i, pl.ds(j, 16)] = jnp.zeros((16,), jnp.float32)

        @pl.loop(0, r, step=ZR)
        def _(rr):
            pltpu.sync_copy(zbuf, accum.at[pl.ds(s * r + rr, ZR)])

        plsc.subcore_barrier()

        base = wid * ep

        for b in range(nbuf - 1):
            pltpu.sync_copy(src_hbm.at[pl.ds(base + b * K, K)], sidx[b])
            pltpu.sync_copy(dst_hbm.at[pl.ds(base + b * K, K)], didx[b])
            pltpu.async_copy(hp_hbm.at[sidx[b]], gbuf[b], sem[b])

        @pl.loop(0, ep, step=nbuf * K)
        def _(t):
            for b in range(nbuf):
                cur = t + b * K
                pre = cur + (nbuf - 1) * K
                sn = (b + nbuf - 1) % nbuf

                @pl.when(pre < ep)
                def _(pre=pre, sn=sn):
                    pltpu.sync_copy(src_hbm.at[pl.ds(base + pre, K)], sidx[sn])
                    pltpu.sync_copy(dst_hbm.at[pl.ds(base + pre, K)], didx[sn])
                    pltpu.async_copy(hp_hbm.at[sidx[sn]], gbuf[sn], sem[sn])

                pltpu.make_async_copy(hp_hbm.at[sidx[b]], gbuf[b], sem[b]).wait()
                pltpu.sync_copy(gbuf[b], accum.at[didx[b]], add=True)

        plsc.subcore_barrier()
        pltpu.sync_copy(accum.at[pl.ds(s * r, r)], out_hbm.at[c].at[pl.ds(s * r, r)])

    return agg_kernel(hp, src_f, dst_f)


def _dinv_from(dp_ref):
    deg = dp_ref[0, :, :1] + dp_ref[1, :, :1]
    return jnp.where(deg > 0, lax.rsqrt(deg), 0.0)


def _tc_k1a(x_pad, W1, blk):
    n_pad, d_in = x_pad.shape
    d_hid = W1.shape[1]

    def body(x_ref, w_ref, o_ref):
        o_ref[...] = jnp.dot(x_ref[...], w_ref[...],
                             preferred_element_type=jnp.float32)

    return pl.pallas_call(
        body,
        grid=(n_pad // blk,),
        in_specs=[
            pl.BlockSpec((blk, d_in), lambda i: (i, 0)),
            pl.BlockSpec((d_in, d_hid), lambda i: (0, 0)),
        ],
        out_specs=pl.BlockSpec((blk, d_hid), lambda i: (i, 0)),
        out_shape=jax.ShapeDtypeStruct((n_pad, d_hid), jnp.float32),
    )(x_pad, W1)


def _tc_k1b(h0, degp, blk):
    n_pad, d_hid = h0.shape

    def body(h_ref, dp_ref, o_ref):
        o_ref[...] = h_ref[...] * _dinv_from(dp_ref)

    return pl.pallas_call(
        body,
        grid=(n_pad // blk,),
        in_specs=[
            pl.BlockSpec((blk, d_hid), lambda i: (i, 0)),
            pl.BlockSpec((NC, blk, 16), lambda i: (0, i, 0)),
        ],
        out_specs=pl.BlockSpec((blk, d_hid), lambda i: (i, 0)),
        out_shape=jax.ShapeDtypeStruct((n_pad, d_hid), jnp.float32),
    )(h0, degp)


def _tc_k2(p, degp, b1, W2, blk):
    n_pad, d_hid = p.shape[1], p.shape[2]
    d_emb = W2.shape[1]

    def body(p_ref, dp_ref, b_ref, w_ref, o_ref):
        dinv = _dinv_from(dp_ref)
        h1 = jnp.maximum((p_ref[0] + p_ref[1]) * dinv + b_ref[...], 0.0)
        o_ref[...] = jnp.dot(h1, w_ref[...],
                             preferred_element_type=jnp.float32) * dinv

    return pl.pallas_call(
        body,
        grid=(n_pad // blk,),
        in_specs=[
            pl.BlockSpec((NC, blk, d_hid), lambda i: (0, i, 0)),
            pl.BlockSpec((NC, blk, 16), lambda i: (0, i, 0)),
            pl.BlockSpec((1, d_hid), lambda i: (0, 0)),
            pl.BlockSpec((d_hid, d_emb), lambda i: (0, 0)),
        ],
        out_specs=pl.BlockSpec((blk, d_emb), lambda i: (i, 0)),
        out_shape=jax.ShapeDtypeStruct((n_pad, d_emb), jnp.float32),
    )(p, degp, b1.reshape(1, -1), W2)


def _tc_k3(q, degp, b2, Wc_pad, bc_pad, blk):
    n_pad, d_emb = q.shape[1], q.shape[2]
    n_out = Wc_pad.shape[1]

    def body(q_ref, dp_ref, b_ref, w_ref, bc_ref, o_ref):
        dinv = _dinv_from(dp_ref)
        h2 = jnp.maximum((q_ref[0] + q_ref[1]) * dinv + b_ref[...], 0.0)
        o_ref[...] = jnp.dot(h2, w_ref[...],
                             preferred_element_type=jnp.float32) + bc_ref[...]

    return pl.pallas_call(
        body,
        grid=(n_pad // blk,),
        in_specs=[
            pl.BlockSpec((NC, blk, d_emb), lambda i: (0, i, 0)),
            pl.BlockSpec((NC, blk, 16), lambda i: (0, i, 0)),
            pl.BlockSpec((1, d_emb), lambda i: (0, 0)),
            pl.BlockSpec((d_emb, n_out), lambda i: (0, 0)),
            pl.BlockSpec((1, n_out), lambda i: (0, 0)),
        ],
        out_specs=pl.BlockSpec((blk, n_out), lambda i: (i, 0)),
        out_shape=jax.ShapeDtypeStruct((n_pad, n_out), jnp.float32),
    )(q, degp, b2.reshape(1, -1), Wc_pad, bc_pad)


def kernel(x, edge_index, W1, b1, W2, b2, Wc, bc):
    n, d_in = x.shape
    d_hid = W1.shape[1]
    d_emb = W2.shape[1]
    n_cls = Wc.shape[1]
    e = edge_index.shape[1]
    e_tot = e + n

    ep = -(-e_tot // (NW * 2 * K)) * 2 * K
    e_pad = ep * NW
    nch = ep // K
    n_pad = -(-(n + 1) // (NS * ZR)) * (NS * ZR)

    idt = edge_index.dtype
    loop_idx = jnp.arange(n, dtype=idt)
    pad_idx = n + jnp.arange(e_pad - e_tot, dtype=idt) % (n_pad - n)
    src_f = jnp.concatenate([edge_index[0], loop_idx, pad_idx])
    dst_f = jnp.concatenate([edge_index[1], loop_idx, pad_idx])
    dst_r = dst_f.reshape(NW, nch, K)

    x_pad = jnp.pad(x.astype(jnp.float32), ((0, n_pad - n), (0, 0)))
    wc_pad = jnp.pad(Wc, ((0, 0), (0, 128 - n_cls)))
    bc_pad = jnp.pad(bc, (0, 128 - n_cls)).reshape(1, -1)

    blk = 2048
    h0 = _tc_k1a(x_pad, W1, blk)
    degp = _sc_degree(dst_r, n_pad)
    hp = _tc_k1b(h0, degp, blk)
    p = _sc_aggregate(hp, src_f, dst_f, n_pad, d_hid, 2)
    h2p = _tc_k2(p, degp, b1, W2, blk)
    q = _sc_aggregate(h2p, src_f, dst_f, n_pad, d_emb, 2)
    outp = _tc_k3(q, degp, b2, wc_pad, bc_pad, blk)
    return outp[:n, :n_cls]
